# Initial kernel scaffold; baseline (speedup 1.0000x reference)
import jax
import jax.numpy as jnp
from jax.experimental import pallas as pl


def kernel(x, row, col, value, spiral_indices, dw_weight, pw_weight, pw_bias):
    raise NotImplementedError("write your pallas kernel here")


# speedup vs baseline: 4.1534x; 4.1534x over previous
import functools

import jax
import jax.numpy as jnp
from jax import lax
from jax.experimental import pallas as pl
from jax.experimental.pallas import tpu as pltpu
from jax.experimental.pallas import tpu_sc as plsc

N_IN = 12500
N_OUT = 50000
C = 128
L = 9
NNZ = 3 * N_OUT

NC = 2
NS = 16
LN = 16

GRP = 128
EPT = NNZ // NS
LOADN = 9384
EBUF = 9392
WINS = EBUF // LN
ACAP = 9600
NQ = 4
CH = 6272
N_OUT_P = 2 * NQ * CH
ZROWS = CH // NS

N_IN_P = 12512
VPT = 1568
VG = 32
NVG = VPT // VG


@functools.cache
def _mesh():
  return plsc.VectorSubcoreMesh(
      core_axis_name="c", subcore_axis_name="s", num_cores=NC, num_subcores=NS)


def _stage1_body(x_hbm, row_hbm, col_hbm, val_hbm, out_hbm,
                 shared, row_v, col_v, val_v, arena,
                 grows, cstage, lstage, vstage, sem):
  c = lax.axis_index("c")
  s = lax.axis_index("s")
  base = s * EPT
  abase = jnp.minimum((base // 8) * 8, NNZ - LOADN)
  abase = pl.multiple_of(abase, 8)
  pltpu.sync_copy(row_hbm.at[pl.ds(abase, LOADN)], row_v.at[pl.ds(0, LOADN)])
  pltpu.sync_copy(col_hbm.at[pl.ds(abase, LOADN)], col_v.at[pl.ds(0, LOADN)])
  pltpu.sync_copy(val_hbm.at[pl.ds(abase, LOADN)], val_v.at[pl.ds(0, LOADN)])
  lo_e = base - abase
  hi_e = lo_e + EPT

  zi = jnp.zeros((LN,), jnp.int32)
  zf = jnp.zeros((LN,), jnp.float32)
  iota = lax.iota(jnp.int32, LN)
  scbase = c * (NQ * CH)

  def prefill(i, _):
    arena[pl.ds(i * LN, LN)] = zi
    return 0
  lax.fori_loop(0, ACAP // LN, prefill, 0)

  def count(i, carry):
    lidx = i * LN + iota
    valid = (lidx >= lo_e) & (lidx < hi_e)
    r16 = row_v[pl.ds(i * LN, LN)]
    rloc = r16 - scbase
    out = []
    for q in range(NQ):
      inq = valid & (rloc >= q * CH) & (rloc < (q + 1) * CH)
      out.append(carry[q] + inq.astype(jnp.int32))
    return tuple(out)
  accs = lax.fori_loop(0, WINS, count, (jnp.zeros((LN,), jnp.int32),) * NQ)
  counts = [jnp.sum(a) for a in accs]
  bases = [jnp.int32(0)]
  for q in range(1, NQ):
    bases.append(bases[q - 1] + counts[q - 1])

  def bucket(i, carry):
    lidx = i * LN + iota
    valid = (lidx >= lo_e) & (lidx < hi_e)
    r16 = row_v[pl.ds(i * LN, LN)]
    rloc = r16 - scbase
    out = []
    for q in range(NQ):
      inq = valid & (rloc >= q * CH) & (rloc < (q + 1) * CH)
      cs = plsc.cumsum(inq.astype(jnp.int32))
      pos = bases[q] + carry[q] + cs - 1
      plsc.store_scatter(arena, [pos], lidx, mask=inq)
      out.append(carry[q] + cs[LN - 1])
    return tuple(out)
  lax.fori_loop(0, WINS, bucket, (jnp.int32(0),) * NQ)

  for q in range(NQ):
    def zrow(r, _):
      for j in range(C // LN):
        grows[r, pl.ds(j * LN, LN)] = zf
      return 0
    lax.fori_loop(0, GRP, zrow, 0)
    zbase = s * ZROWS
    for k in range(ZROWS // GRP):
      pltpu.sync_copy(grows, shared.at[pl.ds(zbase + k * GRP, GRP)])
    rem = ZROWS % GRP
    if rem:
      pltpu.sync_copy(grows.at[pl.ds(0, rem)],
                      shared.at[pl.ds(zbase + (ZROWS // GRP) * GRP, rem)])
    plsc.subcore_barrier()

    nq = counts[q]
    bq = bases[q]
    lo = scbase + q * CH
    ng = (nq + GRP - 1) // GRP
    def group(g, _):
      gb = bq + g * GRP
      for j in range(GRP // LN):
        off = g * GRP + j * LN + iota
        valid = off < nq
        eidx = arena[pl.ds(gb + j * LN, LN)]
        r16 = plsc.load_gather(row_v, [eidx])
        c16 = plsc.load_gather(col_v, [eidx])
        v16 = plsc.load_gather(val_v, [eidx])
        lr = jnp.where(valid, r16 - lo, 0)
        v16 = jnp.where(valid, v16, 0.0)
        cstage[0, pl.ds(j * LN, LN)] = c16
        lstage[0, pl.ds(j * LN, LN)] = lr
        vstage[pl.ds(j * LN, LN)] = v16
      pltpu.async_copy(x_hbm.at[cstage.at[0]], grows, sem).wait()
      def scale(b, _):
        v16 = vstage[pl.ds(b * LN, LN)]
        for k in range(LN):
          v = v16[k]
          r = b * LN + k
          for j in range(C // LN):
            off2 = pl.ds(j * LN, LN)
            grows[r, off2] = grows[r, off2] * v
        return 0
      lax.fori_loop(0, GRP // LN, scale, 0)
      pltpu.sync_copy(grows, shared.at[lstage.at[0]], add=True)
      return 0
    lax.fori_loop(0, ng, group, 0)
    plsc.subcore_barrier()

    chunk_base = (NQ * c + q) * CH
    pltpu.sync_copy(shared.at[pl.ds(zbase, ZROWS)],
                    out_hbm.at[pl.ds(chunk_base + zbase, ZROWS)])


def _stage2_body(pooled_hbm, sp_hbm, w_hbm, out_hbm,
                 sp_v, w_v, g_v, ob, sstage, sem):
  c = lax.axis_index("c")
  s = lax.axis_index("s")
  wid = s * NC + c
  tbase = jnp.minimum(wid * VPT, N_OUT - VPT)
  tbase = pl.multiple_of(tbase, 8)
  tb9 = pl.multiple_of(tbase * L, 8)
  pltpu.sync_copy(sp_hbm.at[pl.ds(tb9, VPT * L)], sp_v)
  pltpu.sync_copy(w_hbm, w_v)
  iota = lax.iota(jnp.int32, LN)

  def group(g, _):
    vb = g * VG
    for l in range(L):
      for w in range(VG // LN):
        pos16 = (vb + w * LN + iota) * L + l
        sv = plsc.load_gather(sp_v, [pos16])
        sstage[0, pl.ds(w * LN, LN)] = sv
      pltpu.async_copy(pooled_hbm.at[sstage.at[0]], g_v.at[l], sem).wait()
    for j in range(C // LN):
      col = pl.ds(j * LN, LN)
      wl = [w_v[l, col] for l in range(L)]
      def rbody(r, _):
        acc = g_v[0, r, col] * wl[0]
        for l in range(1, L):
          acc = acc + g_v[l, r, col] * wl[l]
        ob[r, col] = acc
        return 0
      lax.fori_loop(0, VG, rbody, 0)
    pltpu.sync_copy(ob, out_hbm.at[pl.ds(tbase + vb, VG)])
    return 0
  lax.fori_loop(0, NVG, group, 0)


def _stage3_block(dw_ref, w_ref, b_ref, o_ref):
  acc = jnp.dot(dw_ref[...], w_ref[...], preferred_element_type=jnp.float32)
  o_ref[...] = jnp.maximum(acc + b_ref[...], 0.0)[None]


def kernel(x, row, col, value, spiral_indices, dw_weight, pw_weight, pw_bias):
  row = row.astype(jnp.int32)
  col = col.astype(jnp.int32)
  spf = spiral_indices.astype(jnp.int32).reshape(-1)
  xpad = jnp.concatenate([x[0], jnp.zeros((N_IN_P - N_IN, C), jnp.float32)])
  wpad = jnp.concatenate([dw_weight, jnp.zeros((16 - L, C), jnp.float32)])

  stage1 = pl.kernel(
      _stage1_body,
      out_type=jax.ShapeDtypeStruct((N_OUT_P, C), jnp.float32),
      mesh=_mesh(),
      compiler_params=pltpu.CompilerParams(needs_layout_passes=False),
      scratch_types=[
          pltpu.VMEM_SHARED((CH, C), jnp.float32),
          pltpu.VMEM((EBUF,), jnp.int32),
          pltpu.VMEM((EBUF,), jnp.int32),
          pltpu.VMEM((EBUF,), jnp.float32),
          pltpu.VMEM((ACAP,), jnp.int32),
          pltpu.VMEM((GRP, C), jnp.float32),
          pltpu.VMEM((1, GRP), jnp.int32),
          pltpu.VMEM((1, GRP), jnp.int32),
          pltpu.VMEM((GRP,), jnp.float32),
          pltpu.SemaphoreType.DMA,
      ],
  )
  pooled = stage1(xpad, row, col, value)

  stage2 = pl.kernel(
      _stage2_body,
      out_type=jax.ShapeDtypeStruct((N_OUT, C), jnp.float32),
      mesh=_mesh(),
      compiler_params=pltpu.CompilerParams(needs_layout_passes=False),
      scratch_types=[
          pltpu.VMEM((VPT * L,), jnp.int32),
          pltpu.VMEM((16, C), jnp.float32),
          pltpu.VMEM((L, VG, C), jnp.float32),
          pltpu.VMEM((VG, C), jnp.float32),
          pltpu.VMEM((1, VG), jnp.int32),
          pltpu.SemaphoreType.DMA,
      ],
  )
  dw = stage2(pooled, spf, wpad)

  y = pl.pallas_call(
      _stage3_block,
      grid=(N_OUT // 400,),
      in_specs=[
          pl.BlockSpec((400, C), lambda i: (i, 0)),
          pl.BlockSpec((C, C), lambda i: (0, 0)),
          pl.BlockSpec((1, C), lambda i: (0, 0)),
      ],
      out_specs=pl.BlockSpec((1, 400, C), lambda i: (0, i, 0)),
      out_shape=jax.ShapeDtypeStruct((1, N_OUT, C), jnp.float32),
  )(dw, pw_weight, pw_bias.reshape(1, C))

  return y


# speedup vs baseline: 7.5875x; 1.8268x over previous
import functools

import jax
import jax.numpy as jnp
from jax import lax
from jax.experimental import pallas as pl
from jax.experimental.pallas import tpu as pltpu
from jax.experimental.pallas import tpu_sc as plsc

N_IN = 12500
N_OUT = 50000
C = 128
L = 9
NNZ = 3 * N_OUT

NC = 2
NS = 16
LN = 16

GRP = 128
EPT = NNZ // NS
LOADN = 9384
EBUF = 9392
WINS = EBUF // LN
ACAP = 9984
NQ = 4
CH = 6272
N_OUT_P = 2 * NQ * CH
ZROWS = CH // NS

N_IN_P = 12512
VPT = 1568
VG = 32
NVG = VPT // VG


@functools.cache
def _mesh():
  return plsc.VectorSubcoreMesh(
      core_axis_name="c", subcore_axis_name="s", num_cores=NC, num_subcores=NS)


def _stage1_body(x_hbm, row_hbm, col_hbm, val_hbm, out_hbm,
                 shared, row_v, col_v, val_v, arena,
                 gr0, gr1, cst0, lst0, vst0, cst1, lst1, vst1,
                 semg0, semg1, sems0, sems1):
  c = lax.axis_index("c")
  s = lax.axis_index("s")
  base = s * EPT
  abase = jnp.minimum((base // 8) * 8, NNZ - LOADN)
  abase = pl.multiple_of(abase, 8)
  pltpu.sync_copy(row_hbm.at[pl.ds(abase, LOADN)], row_v.at[pl.ds(0, LOADN)])
  pltpu.sync_copy(col_hbm.at[pl.ds(abase, LOADN)], col_v.at[pl.ds(0, LOADN)])
  pltpu.sync_copy(val_hbm.at[pl.ds(abase, LOADN)], val_v.at[pl.ds(0, LOADN)])
  lo_e = base - abase
  hi_e = lo_e + EPT

  zi = jnp.zeros((LN,), jnp.int32)
  zf = jnp.zeros((LN,), jnp.float32)
  iota = lax.iota(jnp.int32, LN)
  scbase = c * (NQ * CH)

  def prefill(i, _):
    arena[pl.ds(i * LN, LN)] = zi
    return 0
  lax.fori_loop(0, ACAP // LN, prefill, 0)

  def count(i, carry):
    lidx = i * LN + iota
    valid = (lidx >= lo_e) & (lidx < hi_e)
    r16 = row_v[pl.ds(i * LN, LN)]
    rloc = r16 - scbase
    out = []
    for q in range(NQ):
      inq = valid & (rloc >= q * CH) & (rloc < (q + 1) * CH)
      out.append(carry[q] + inq.astype(jnp.int32))
    return tuple(out)
  accs = lax.fori_loop(0, WINS, count, (jnp.zeros((LN,), jnp.int32),) * NQ)
  counts = [jnp.sum(a) for a in accs]
  bases = [jnp.int32(0)]
  for q in range(1, NQ):
    bases.append(bases[q - 1] + counts[q - 1])

  def bucket(i, carry):
    lidx = i * LN + iota
    valid = (lidx >= lo_e) & (lidx < hi_e)
    r16 = row_v[pl.ds(i * LN, LN)]
    rloc = r16 - scbase
    out = []
    for q in range(NQ):
      inq = valid & (rloc >= q * CH) & (rloc < (q + 1) * CH)
      cs = plsc.cumsum(inq.astype(jnp.int32))
      pos = bases[q] + carry[q] + cs - 1
      plsc.store_scatter(arena, [pos], lidx, mask=inq)
      out.append(carry[q] + cs[LN - 1])
    return tuple(out)
  lax.fori_loop(0, WINS, bucket, (jnp.int32(0),) * NQ)

  for q in range(NQ):
    def zrow(r, _):
      for j in range(C // LN):
        gr0[r, pl.ds(j * LN, LN)] = zf
      return 0
    lax.fori_loop(0, GRP, zrow, 0)
    zbase = s * ZROWS
    for k in range(ZROWS // GRP):
      pltpu.sync_copy(gr0, shared.at[pl.ds(zbase + k * GRP, GRP)])
    rem = ZROWS % GRP
    if rem:
      pltpu.sync_copy(gr0.at[pl.ds(0, rem)],
                      shared.at[pl.ds(zbase + (ZROWS // GRP) * GRP, rem)])
    plsc.subcore_barrier()

    nq = counts[q]
    bq = bases[q]
    lo = scbase + q * CH

    def stage_idx(g, cst, lst, vst):
      gb = bq + g * GRP
      for j in range(GRP // LN):
        off = g * GRP + j * LN + iota
        valid = off < nq
        eidx = arena[pl.ds(gb + j * LN, LN)]
        r16 = plsc.load_gather(row_v, [eidx])
        c16 = plsc.load_gather(col_v, [eidx])
        v16 = plsc.load_gather(val_v, [eidx])
        lr = jnp.where(valid, r16 - lo, 0)
        v16 = jnp.where(valid, v16, 0.0)
        cst[0, pl.ds(j * LN, LN)] = c16
        lst[0, pl.ds(j * LN, LN)] = lr
        vst[pl.ds(j * LN, LN)] = v16

    def scale_buf(gr, vst):
      def scale(b, _):
        v16 = vst[pl.ds(b * LN, LN)]
        for k in range(LN):
          v = v16[k]
          r = b * LN + k
          for j in range(C // LN):
            off2 = pl.ds(j * LN, LN)
            gr[r, off2] = gr[r, off2] * v
        return 0
      lax.fori_loop(0, GRP // LN, scale, 0)

    def drain_gather(gr, sem):
      pltpu.make_async_copy(x_hbm.at[pl.ds(0, GRP)], gr, sem).wait()

    def drain_scatter(gr, sem):
      pltpu.make_async_copy(gr, shared.at[pl.ds(0, GRP)], sem).wait()

    ngp = jnp.maximum((nq + 2 * GRP - 1) // (2 * GRP), 1)
    stage_idx(0, cst0, lst0, vst0)
    pltpu.async_copy(x_hbm.at[cst0.at[0]], gr0, semg0)
    def pair(pp, _):
      ga = pp * 2
      drain_gather(gr0, semg0)
      @pl.when(pp > 0)
      def _():
        drain_scatter(gr1, sems1)
      stage_idx(ga + 1, cst1, lst1, vst1)
      pltpu.async_copy(x_hbm.at[cst1.at[0]], gr1, semg1)
      scale_buf(gr0, vst0)
      pltpu.async_copy(gr0, shared.at[lst0.at[0]], sems0, add=True)
      drain_gather(gr1, semg1)
      drain_scatter(gr0, sems0)
      stage_idx(ga + 2, cst0, lst0, vst0)
      pltpu.async_copy(x_hbm.at[cst0.at[0]], gr0, semg0)
      scale_buf(gr1, vst1)
      pltpu.async_copy(gr1, shared.at[lst1.at[0]], sems1, add=True)
      return 0
    lax.fori_loop(0, ngp, pair, 0)
    drain_gather(gr0, semg0)
    drain_scatter(gr1, sems1)
    plsc.subcore_barrier()

    chunk_base = (NQ * c + q) * CH
    pltpu.sync_copy(shared.at[pl.ds(zbase, ZROWS)],
                    out_hbm.at[pl.ds(chunk_base + zbase, ZROWS)])


def _stage2_body(pooled_hbm, sp_hbm, w_hbm, out_hbm,
                 sp_v, w_v, g0, g1, ob, st0, st1, sem0, sem1):
  c = lax.axis_index("c")
  s = lax.axis_index("s")
  wid = s * NC + c
  tbase = jnp.minimum(wid * VPT, N_OUT - VPT)
  tbase = pl.multiple_of(tbase, 8)
  tb9 = pl.multiple_of(tbase * L, 8)
  pltpu.sync_copy(sp_hbm.at[pl.ds(tb9, VPT * L)], sp_v)
  pltpu.sync_copy(w_hbm, w_v)
  iota = lax.iota(jnp.int32, LN)

  def fire(g, st, gbuf, sem):
    vb = g * VG
    for l in range(L):
      for w in range(VG // LN):
        pos16 = (vb + w * LN + iota) * L + l
        st[0, pl.ds(l * VG + w * LN, LN)] = plsc.load_gather(sp_v, [pos16])
    for l in range(L):
      pltpu.async_copy(pooled_hbm.at[st.at[0, pl.ds(l * VG, VG)]],
                       gbuf.at[l], sem)

  def drain(gbuf, sem):
    for l in range(L):
      pltpu.make_async_copy(pooled_hbm.at[pl.ds(0, VG)], gbuf.at[l],
                            sem).wait()

  def compute(g, gbuf):
    for j in range(C // LN):
      col = pl.ds(j * LN, LN)
      wl = [w_v[l, col] for l in range(L)]
      def rbody(r, _):
        acc = gbuf[0, r, col] * wl[0]
        for l in range(1, L):
          acc = acc + gbuf[l, r, col] * wl[l]
        ob[r, col] = acc
        return 0
      lax.fori_loop(0, VG, rbody, 0)
    pltpu.sync_copy(ob, out_hbm.at[pl.ds(tbase + g * VG, VG)])

  fire(0, st0, g0, sem0)
  def pair(pp, _):
    ga = pp * 2
    drain(g0, sem0)
    fire(ga + 1, st1, g1, sem1)
    compute(ga, g0)
    drain(g1, sem1)
    fire(ga + 2, st0, g0, sem0)
    compute(ga + 1, g1)
    return 0
  lax.fori_loop(0, (NVG - 1) // 2, pair, 0)
  drain(g0, sem0)
  compute(NVG - 1, g0)


def _stage3_block(dw_ref, w_ref, b_ref, o_ref):
  acc = jnp.dot(dw_ref[...], w_ref[...], preferred_element_type=jnp.float32)
  o_ref[...] = jnp.maximum(acc + b_ref[...], 0.0)[None]


def kernel(x, row, col, value, spiral_indices, dw_weight, pw_weight, pw_bias):
  row = row.astype(jnp.int32)
  col = col.astype(jnp.int32)
  spf = spiral_indices.astype(jnp.int32).reshape(-1)
  xpad = jnp.concatenate([x[0], jnp.zeros((N_IN_P - N_IN, C), jnp.float32)])
  wpad = jnp.concatenate([dw_weight, jnp.zeros((16 - L, C), jnp.float32)])

  stage1 = pl.kernel(
      _stage1_body,
      out_type=jax.ShapeDtypeStruct((N_OUT_P, C), jnp.float32),
      mesh=_mesh(),
      compiler_params=pltpu.CompilerParams(needs_layout_passes=False),
      scratch_types=[
          pltpu.VMEM_SHARED((CH, C), jnp.float32),
          pltpu.VMEM((EBUF,), jnp.int32),
          pltpu.VMEM((EBUF,), jnp.int32),
          pltpu.VMEM((EBUF,), jnp.float32),
          pltpu.VMEM((ACAP,), jnp.int32),
          pltpu.VMEM((GRP, C), jnp.float32),
          pltpu.VMEM((GRP, C), jnp.float32),
          pltpu.VMEM((1, GRP), jnp.int32),
          pltpu.VMEM((1, GRP), jnp.int32),
          pltpu.VMEM((GRP,), jnp.float32),
          pltpu.VMEM((1, GRP), jnp.int32),
          pltpu.VMEM((1, GRP), jnp.int32),
          pltpu.VMEM((GRP,), jnp.float32),
          pltpu.SemaphoreType.DMA,
          pltpu.SemaphoreType.DMA,
          pltpu.SemaphoreType.DMA,
          pltpu.SemaphoreType.DMA,
      ],
  )
  pooled = stage1(xpad, row, col, value)

  stage2 = pl.kernel(
      _stage2_body,
      out_type=jax.ShapeDtypeStruct((N_OUT, C), jnp.float32),
      mesh=_mesh(),
      compiler_params=pltpu.CompilerParams(needs_layout_passes=False),
      scratch_types=[
          pltpu.VMEM((VPT * L,), jnp.int32),
          pltpu.VMEM((16, C), jnp.float32),
          pltpu.VMEM((L, VG, C), jnp.float32),
          pltpu.VMEM((L, VG, C), jnp.float32),
          pltpu.VMEM((VG, C), jnp.float32),
          pltpu.VMEM((1, L * VG), jnp.int32),
          pltpu.VMEM((1, L * VG), jnp.int32),
          pltpu.SemaphoreType.DMA,
          pltpu.SemaphoreType.DMA,
      ],
  )
  dw = stage2(pooled, spf, wpad)

  y = pl.pallas_call(
      _stage3_block,
      grid=(N_OUT // 400,),
      in_specs=[
          pl.BlockSpec((400, C), lambda i: (i, 0)),
          pl.BlockSpec((C, C), lambda i: (0, 0)),
          pl.BlockSpec((1, C), lambda i: (0, 0)),
      ],
      out_specs=pl.BlockSpec((1, 400, C), lambda i: (0, i, 0)),
      out_shape=jax.ShapeDtypeStruct((1, N_OUT, C), jnp.float32),
  )(dw, pw_weight, pw_bias.reshape(1, C))

  return y
